# baseline (speedup 1.0000x reference)
import functools

import jax
import jax.numpy as jnp
from jax import lax
from jax.experimental import pallas as pl
from jax.experimental.pallas import tpu as pltpu
from jax.experimental.pallas import tpu_sc as plsc

BATCH = 32
SEQ = 256
TOK = BATCH * SEQ
D = 1024
V1 = 1000
EPS = 1e-3
NC = 2
NS = 16
NW = NC * NS
NSPLIT = 2
HTOK = TOK // NSPLIT
HBATCH = BATCH // NSPLIT
TPW = HTOK // NW
C = 32
NCHUNK = TPW // C
NBUF = 3


def _gather_kernel(x0_h, emb0_h, content_h, idx_v, r_v, gsem, osem):
    wid = lax.axis_index("s") * NC + lax.axis_index("c")
    base = wid * TPW

    pltpu.sync_copy(x0_h.at[pl.ds(base, TPW)], idx_v)

    def start(ci, b):
        return pltpu.async_copy(emb0_h.at[idx_v.at[pl.ds(ci * C, C)]],
                                r_v.at[b], gsem[b])

    def drain(ci, b, gcp):
        gcp.wait()
        return pltpu.async_copy(r_v.at[b], content_h.at[pl.ds(base + ci * C, C)],
                                osem[b])

    gcps = {ci: start(ci, ci % NBUF) for ci in range(min(NBUF, NCHUNK))}
    ocps = {}
    for ci in range(NCHUNK):
        b = ci % NBUF
        ocps[ci] = drain(ci, b, gcps.pop(ci))
        nx = ci + NBUF
        if nx < NCHUNK:
            ocps.pop(nx - NBUF).wait()
            gcps[nx] = start(nx, nx % NBUF)
    for ocp in ocps.values():
        ocp.wait()


def _tim_body(pos_ref, tim_ref):
    tim_ref[...] = pos_ref[...]


def _ln_body(*refs):
    if len(refs) == 8:
        refs = refs[1:]
    content_ref, x1_ref, emb1_ref, pos_ref, a2_ref, b2_ref, out_ref = refs
    x1b = x1_ref[0, 0, :]
    iota = lax.broadcasted_iota(jnp.int32, (SEQ, V1), 1)
    onehot = (x1b[:, None] == iota).astype(jnp.bfloat16)
    e1 = emb1_ref[...].astype(jnp.bfloat16)
    emb1_rows = jnp.dot(onehot, e1, preferred_element_type=jnp.float32)
    z = content_ref[...] + emb1_rows + pos_ref[...]
    mu = jnp.mean(z, axis=-1, keepdims=True)
    zc = z - mu
    var = jnp.sum(zc * zc, axis=-1, keepdims=True) * (1.0 / (D - 1))
    y = lax.rsqrt(var + 1e-30)
    for _ in range(3):
        y = y * (1.5 - 0.5 * var * y * y)
    sigma = var * y
    den = sigma + EPS
    r = 1.0 / den
    for _ in range(3):
        r = r * (2.0 - den * r)
    out_ref[...] = zc * r * a2_ref[...] + b2_ref[...]


def kernel(x0, x1, emb0, emb1, position_table, a_2, b_2):
    mesh = plsc.VectorSubcoreMesh(core_axis_name="c", subcore_axis_name="s")
    gather = pl.kernel(
        _gather_kernel,
        out_type=jax.ShapeDtypeStruct((HTOK, D), jnp.float32),
        mesh=mesh,
        scratch_types=[
            pltpu.VMEM((TPW,), jnp.int32),
            pltpu.VMEM((NBUF, C, D), jnp.float32),
            [pltpu.SemaphoreType.DMA] * NBUF,
            [pltpu.SemaphoreType.DMA] * NBUF,
        ],
    )
    x0 = x0.astype(jnp.int32)
    contents = [gather(x0[h * HTOK:(h + 1) * HTOK], emb0)
                for h in range(NSPLIT)]

    pos = position_table[:SEQ]
    tim = pl.pallas_call(
        _tim_body,
        grid=(BATCH,),
        in_specs=[pl.BlockSpec((SEQ, D), lambda i: (0, 0))],
        out_specs=pl.BlockSpec((SEQ, D), lambda i: (i, 0)),
        out_shape=jax.ShapeDtypeStruct((TOK, D), jnp.float32),
    )(pos)

    x1r = x1.astype(jnp.int32).reshape(BATCH, 1, SEQ)
    out = None
    for h in range(NSPLIT):
        main_specs = [
            pl.BlockSpec((SEQ, D), lambda i: (i, 0)),
            pl.BlockSpec((1, 1, SEQ), lambda i, h=h: (h * HBATCH + i, 0, 0)),
            pl.BlockSpec((V1, D), lambda i: (0, 0)),
            pl.BlockSpec((SEQ, D), lambda i: (0, 0)),
            pl.BlockSpec((D,), lambda i: (0,)),
            pl.BlockSpec((D,), lambda i: (0,)),
        ]
        prev_spec = [] if h == 0 else [pl.BlockSpec(memory_space=pl.ANY)]
        ln = pl.pallas_call(
            _ln_body,
            grid=(HBATCH,),
            in_specs=prev_spec + main_specs,
            out_specs=pl.BlockSpec((SEQ, D), lambda i, h=h: (h * HBATCH + i, 0)),
            out_shape=jax.ShapeDtypeStruct((TOK, D), jnp.float32),
            input_output_aliases={} if h == 0 else {0: 0},
        )
        prev = () if h == 0 else (out,)
        out = ln(*prev, contents[h], x1r, emb1, pos, a_2, b_2)
    return (out, tim)
